# baseline (speedup 1.0000x reference)
import functools

import jax
import jax.numpy as jnp
from jax import lax
from jax.experimental import pallas as pl
from jax.experimental.pallas import tpu as pltpu
from jax.experimental.pallas import tpu_sc as plsc

N_NODES = 10000
N_EDGES = 320000
IN_DIM = 128
HIDDEN = 64
OUT_DIM = 8
N_GRAPHS = 64

NC, NS = 2, 16
NW = NC * NS
CH = 125
CPT = 80
ROWS_PAD = 10240
RPT = ROWS_PAD // NS

ROW_BLK = 1000
N_BLK = N_NODES // ROW_BLK
PAD_BLK = 1024
N_PAD_BLK = ROWS_PAD // PAD_BLK

_sc_mesh = plsc.VectorSubcoreMesh(core_axis_name="c", subcore_axis_name="s")
_sc_params = pltpu.CompilerParams(use_tc_tiling_on_sc=False)


def _deg_body(dst_hbm, zeros_hbm, ones_hbm, degp_hbm,
              dstv, valv, acc, sem):
    c = lax.axis_index("c")
    s = lax.axis_index("s")
    w = c * NS + s
    pltpu.sync_copy(zeros_hbm, acc.at[pl.ds(s * RPT, RPT)])
    pltpu.sync_copy(dst_hbm.at[w], dstv)
    pltpu.sync_copy(ones_hbm, valv)
    plsc.subcore_barrier()

    def step(j, carry):
        pltpu.sync_copy(valv, acc.at[dstv.at[j]], add=True)
        return carry

    lax.fori_loop(0, CPT, step, 0, unroll=4)
    plsc.subcore_barrier()
    pltpu.sync_copy(acc.at[pl.ds(s * RPT, RPT)],
                    degp_hbm.at[c, pl.ds(s * RPT, RPT)])


_deg_kernel = functools.partial(
    pl.kernel,
    out_type=jax.ShapeDtypeStruct((NC, ROWS_PAD, 16), jnp.float32),
    mesh=_sc_mesh,
    scratch_types=[
        pltpu.VMEM((CPT, CH), jnp.int32),
        pltpu.VMEM((CH, 16), jnp.float32),
        pltpu.VMEM_SHARED((ROWS_PAD, 16), jnp.float32),
        pltpu.SemaphoreType.DMA,
    ],
    compiler_params=_sc_params,
)(_deg_body)


def _edge_phase(hp_s, acc, srcv, dstv, rows_a, rows_b, sem_a, sem_b):
    ra = rows_a.at[pl.ds(0, CH)]
    rb = rows_b.at[pl.ds(0, CH)]
    pltpu.async_copy(hp_s.at[srcv.at[0]], ra, sem_a)

    def step(jj, carry):
        c0 = 2 * jj
        c1 = c0 + 1
        c2 = jnp.minimum(c0 + 2, CPT - 1)
        pltpu.make_async_copy(hp_s.at[srcv.at[c0]], ra, sem_a).wait()
        pltpu.async_copy(hp_s.at[srcv.at[c1]], rb, sem_b)
        pltpu.sync_copy(ra, acc.at[dstv.at[c0]], add=True)
        pltpu.make_async_copy(hp_s.at[srcv.at[c1]], rb, sem_b).wait()
        pltpu.async_copy(hp_s.at[srcv.at[c2]], ra, sem_a)
        pltpu.sync_copy(rb, acc.at[dstv.at[c1]], add=True)
        return carry

    lax.fori_loop(0, CPT // 2, step, 0)
    pltpu.make_async_copy(hp_s.at[srcv.at[CPT - 1]], ra, sem_a).wait()


def _prop_body(hp_hbm, src_hbm, dst_hbm, zeros_hbm, out_hbm,
                srcv, dstv, rows_a, rows_b, acc, hp_s, sem_a, sem_b):
    c = lax.axis_index("c")
    s = lax.axis_index("s")
    w = c * NS + s
    row0 = s * RPT
    pltpu.sync_copy(zeros_hbm, acc.at[pl.ds(row0, RPT)])
    pltpu.sync_copy(src_hbm.at[w], srcv)
    pltpu.sync_copy(dst_hbm.at[w], dstv)
    pltpu.sync_copy(hp_hbm.at[pl.ds(row0, RPT)], hp_s.at[pl.ds(row0, RPT)])
    plsc.subcore_barrier()
    _edge_phase(hp_s, acc, srcv, dstv, rows_a, rows_b, sem_a, sem_b)
    plsc.subcore_barrier()
    pltpu.sync_copy(acc.at[pl.ds(row0, RPT)],
                    out_hbm.at[c, pl.ds(row0, RPT)])


_prop_kernel = functools.partial(
    pl.kernel,
    out_type=jax.ShapeDtypeStruct((NC, ROWS_PAD, HIDDEN), jnp.float32),
    mesh=_sc_mesh,
    scratch_types=[
        pltpu.VMEM((CPT, CH), jnp.int32),
        pltpu.VMEM((CPT, CH), jnp.int32),
        pltpu.VMEM((CH, HIDDEN), jnp.float32),
        pltpu.VMEM((CH, HIDDEN), jnp.float32),
        pltpu.VMEM_SHARED((ROWS_PAD, HIDDEN), jnp.float32),
        pltpu.VMEM_SHARED((ROWS_PAD, HIDDEN), jnp.float32),
        pltpu.SemaphoreType.DMA,
        pltpu.SemaphoreType.DMA,
    ],
    compiler_params=_sc_params,
)(_prop_body)


def _dinv_from(degp_ref):
    deg = degp_ref[0, :, 0:1] + degp_ref[1, :, 0:1] + 1.0
    return lax.rsqrt(deg)


def _tc_mm1_body(degp_ref, x_ref, w1_ref, hp1_ref):
    hp1_ref[...] = jnp.dot(x_ref[...] * _dinv_from(degp_ref), w1_ref[...],
                           preferred_element_type=jnp.float32)


def _tc_mm1(degp, x, W1):
    return pl.pallas_call(
        _tc_mm1_body,
        grid=(N_BLK,),
        in_specs=[
            pl.BlockSpec((NC, ROW_BLK, 16), lambda i: (0, i, 0)),
            pl.BlockSpec((ROW_BLK, IN_DIM), lambda i: (i, 0)),
            pl.BlockSpec((IN_DIM, HIDDEN), lambda i: (0, 0)),
        ],
        out_specs=pl.BlockSpec((ROW_BLK, HIDDEN), lambda i: (i, 0)),
        out_shape=jax.ShapeDtypeStruct((ROWS_PAD, HIDDEN), jnp.float32),
    )(degp, x, W1)


def _tc_b_body(degp_ref, hp1_ref, sp_ref, b_ref, w2_ref, hp2_ref):
    dinv = _dinv_from(degp_ref)
    h = (sp_ref[0] + sp_ref[1] + hp1_ref[...]) * dinv + b_ref[...]
    h = jnp.maximum(h, 0.0)
    hp2_ref[...] = jnp.dot(h, w2_ref[...],
                           preferred_element_type=jnp.float32) * dinv


def _tc_b(degp, hp1, sp, b1, W2):
    return pl.pallas_call(
        _tc_b_body,
        grid=(N_BLK,),
        in_specs=[
            pl.BlockSpec((NC, ROW_BLK, 16), lambda i: (0, i, 0)),
            pl.BlockSpec((ROW_BLK, HIDDEN), lambda i: (i, 0)),
            pl.BlockSpec((NC, ROW_BLK, HIDDEN), lambda i: (0, i, 0)),
            pl.BlockSpec((1, HIDDEN), lambda i: (0, 0)),
            pl.BlockSpec((HIDDEN, HIDDEN), lambda i: (0, 0)),
        ],
        out_specs=pl.BlockSpec((ROW_BLK, HIDDEN), lambda i: (i, 0)),
        out_shape=jax.ShapeDtypeStruct((ROWS_PAD, HIDDEN), jnp.float32),
    )(degp, hp1, sp, b1, W2)


def _tc_c_body(degp_ref, sp_ref, hp_ref, b_ref, batch_ref, wlin_ref,
               blin_ref, out_ref, pooled_scr, counts_scr):
    i = pl.program_id(0)

    @pl.when(i == 0)
    def _init():
        pooled_scr[...] = jnp.zeros_like(pooled_scr)
        counts_scr[...] = jnp.zeros_like(counts_scr)

    dinv = _dinv_from(degp_ref)
    h = (sp_ref[0] + sp_ref[1] + hp_ref[...]) * dinv + b_ref[...]
    h = jnp.maximum(h, 0.0)
    gids = lax.broadcasted_iota(jnp.int32, (ROW_BLK, N_GRAPHS), 1
                                ).astype(jnp.float32)
    onehot = jnp.where(batch_ref[...] == gids, 1.0, 0.0)
    pooled_scr[...] += lax.dot_general(
        onehot, h, (((0,), (0,)), ((), ())),
        preferred_element_type=jnp.float32)
    counts_scr[...] += lax.dot_general(
        onehot, jnp.ones((ROW_BLK, N_GRAPHS), jnp.float32),
        (((0,), (0,)), ((), ())), preferred_element_type=jnp.float32)

    @pl.when(i == N_BLK - 1)
    def _fin():
        counts = jnp.maximum(counts_scr[...][:, 0:HIDDEN], 1.0)
        pooled = pooled_scr[...] / counts
        out_ref[...] = jnp.dot(pooled, wlin_ref[...],
                               preferred_element_type=jnp.float32) + blin_ref[...]


def _tc_c(degp, sp, hp2, b2, batchf, Wlin, blin):
    return pl.pallas_call(
        _tc_c_body,
        grid=(N_BLK,),
        in_specs=[
            pl.BlockSpec((NC, ROW_BLK, 16), lambda i: (0, i, 0)),
            pl.BlockSpec((NC, ROW_BLK, HIDDEN), lambda i: (0, i, 0)),
            pl.BlockSpec((ROW_BLK, HIDDEN), lambda i: (i, 0)),
            pl.BlockSpec((1, HIDDEN), lambda i: (0, 0)),
            pl.BlockSpec((ROW_BLK, 1), lambda i: (i, 0)),
            pl.BlockSpec((HIDDEN, OUT_DIM), lambda i: (0, 0)),
            pl.BlockSpec((1, OUT_DIM), lambda i: (0, 0)),
        ],
        out_specs=pl.BlockSpec((N_GRAPHS, OUT_DIM), lambda i: (0, 0)),
        out_shape=jax.ShapeDtypeStruct((N_GRAPHS, OUT_DIM), jnp.float32),
        scratch_shapes=[
            pltpu.VMEM((N_GRAPHS, N_GRAPHS), jnp.float32),
            pltpu.VMEM((N_GRAPHS, N_GRAPHS), jnp.float32),
        ],
    )(degp, sp, hp2, b2, batchf, Wlin, blin)


@jax.jit
def kernel(x, edge_index, batch, W1, b1, W2, b2, Wlin, blin):
    srcp = edge_index[0].astype(jnp.int32).reshape(NW, CPT, CH)
    dstp = edge_index[1].astype(jnp.int32).reshape(NW, CPT, CH)

    zeros16 = jnp.zeros((RPT, 16), jnp.float32)
    zeros64 = jnp.zeros((RPT, HIDDEN), jnp.float32)
    ones_rows = jnp.ones((CH, 16), jnp.float32)

    degp = _deg_kernel(dstp, zeros16, ones_rows)
    hp1 = _tc_mm1(degp, x, W1)
    sp1 = _prop_kernel(hp1, srcp, dstp, zeros64)
    hp2 = _tc_b(degp, hp1, sp1, b1.reshape(1, HIDDEN), W2)
    sp2 = _prop_kernel(hp2, srcp, dstp, zeros64)
    batchf = batch.astype(jnp.float32).reshape(N_NODES, 1)
    logits = _tc_c(degp, sp2, hp2, b2.reshape(1, HIDDEN), batchf,
                   Wlin, blin.reshape(1, OUT_DIM))
    return logits


# speedup vs baseline: 1.0099x; 1.0099x over previous
import functools

import jax
import jax.numpy as jnp
from jax import lax
from jax.experimental import pallas as pl
from jax.experimental.pallas import tpu as pltpu
from jax.experimental.pallas import tpu_sc as plsc

N_NODES = 10000
N_EDGES = 320000
IN_DIM = 128
HIDDEN = 64
OUT_DIM = 8
N_GRAPHS = 64

NC, NS = 2, 16
NW = NC * NS
CH = 125
CPT = 80
ROWS_PAD = 10240
RPT = ROWS_PAD // NS

ROW_BLK = 1000
N_BLK = N_NODES // ROW_BLK
PAD_BLK = 1024
N_PAD_BLK = ROWS_PAD // PAD_BLK

_sc_mesh = plsc.VectorSubcoreMesh(core_axis_name="c", subcore_axis_name="s")
_sc_params = pltpu.CompilerParams(use_tc_tiling_on_sc=False)


def _deg_body(dst_hbm, zeros_hbm, ones_hbm, degp_hbm,
              dstv, valv, acc, sem):
    c = lax.axis_index("c")
    s = lax.axis_index("s")
    w = c * NS + s
    pltpu.sync_copy(zeros_hbm, acc.at[pl.ds(s * RPT, RPT)])
    pltpu.sync_copy(dst_hbm.at[w], dstv)
    pltpu.sync_copy(ones_hbm, valv)
    plsc.subcore_barrier()

    def step(j, carry):
        pltpu.sync_copy(valv, acc.at[dstv.at[j]], add=True)
        return carry

    lax.fori_loop(0, CPT, step, 0, unroll=4)
    plsc.subcore_barrier()
    pltpu.sync_copy(acc.at[pl.ds(s * RPT, RPT)],
                    degp_hbm.at[c, pl.ds(s * RPT, RPT)])


_deg_kernel = functools.partial(
    pl.kernel,
    out_type=jax.ShapeDtypeStruct((NC, ROWS_PAD, 8), jnp.float32),
    mesh=_sc_mesh,
    scratch_types=[
        pltpu.VMEM((CPT, CH), jnp.int32),
        pltpu.VMEM((CH, 8), jnp.float32),
        pltpu.VMEM_SHARED((ROWS_PAD, 8), jnp.float32),
        pltpu.SemaphoreType.DMA,
    ],
    compiler_params=_sc_params,
)(_deg_body)


def _edge_phase(hp_s, acc, srcv, dstv, rows_a, rows_b, sem_a, sem_b):
    ra = rows_a.at[pl.ds(0, CH)]
    rb = rows_b.at[pl.ds(0, CH)]
    pltpu.async_copy(hp_s.at[srcv.at[0]], ra, sem_a)

    def step(jj, carry):
        c0 = 2 * jj
        c1 = c0 + 1
        c2 = jnp.minimum(c0 + 2, CPT - 1)
        pltpu.make_async_copy(hp_s.at[srcv.at[c0]], ra, sem_a).wait()
        pltpu.async_copy(hp_s.at[srcv.at[c1]], rb, sem_b)
        pltpu.sync_copy(ra, acc.at[dstv.at[c0]], add=True)
        pltpu.make_async_copy(hp_s.at[srcv.at[c1]], rb, sem_b).wait()
        pltpu.async_copy(hp_s.at[srcv.at[c2]], ra, sem_a)
        pltpu.sync_copy(rb, acc.at[dstv.at[c1]], add=True)
        return carry

    lax.fori_loop(0, CPT // 2, step, 0)
    pltpu.make_async_copy(hp_s.at[srcv.at[CPT - 1]], ra, sem_a).wait()


def _prop_body(hp_hbm, src_hbm, dst_hbm, zeros_hbm, out_hbm,
                srcv, dstv, rows_a, rows_b, acc, hp_s, sem_a, sem_b):
    c = lax.axis_index("c")
    s = lax.axis_index("s")
    w = c * NS + s
    row0 = s * RPT
    pltpu.sync_copy(zeros_hbm, acc.at[pl.ds(row0, RPT)])
    pltpu.sync_copy(src_hbm.at[w], srcv)
    pltpu.sync_copy(dst_hbm.at[w], dstv)
    pltpu.sync_copy(hp_hbm.at[pl.ds(row0, RPT)], hp_s.at[pl.ds(row0, RPT)])
    plsc.subcore_barrier()
    _edge_phase(hp_s, acc, srcv, dstv, rows_a, rows_b, sem_a, sem_b)
    plsc.subcore_barrier()
    pltpu.sync_copy(acc.at[pl.ds(row0, RPT)],
                    out_hbm.at[c, pl.ds(row0, RPT)])


_prop_kernel = functools.partial(
    pl.kernel,
    out_type=jax.ShapeDtypeStruct((NC, ROWS_PAD, HIDDEN), jnp.float32),
    mesh=_sc_mesh,
    scratch_types=[
        pltpu.VMEM((CPT, CH), jnp.int32),
        pltpu.VMEM((CPT, CH), jnp.int32),
        pltpu.VMEM((CH, HIDDEN), jnp.float32),
        pltpu.VMEM((CH, HIDDEN), jnp.float32),
        pltpu.VMEM_SHARED((ROWS_PAD, HIDDEN), jnp.float32),
        pltpu.VMEM_SHARED((ROWS_PAD, HIDDEN), jnp.float32),
        pltpu.SemaphoreType.DMA,
        pltpu.SemaphoreType.DMA,
    ],
    compiler_params=_sc_params,
)(_prop_body)


def _tc_mm1_body(x_ref, w1_ref, mm1_ref):
    mm1_ref[...] = jnp.dot(x_ref[...], w1_ref[...],
                           preferred_element_type=jnp.float32)


def _tc_mm1(x, W1):
    return pl.pallas_call(
        _tc_mm1_body,
        grid=(N_BLK,),
        in_specs=[
            pl.BlockSpec((ROW_BLK, IN_DIM), lambda i: (i, 0)),
            pl.BlockSpec((IN_DIM, HIDDEN), lambda i: (0, 0)),
        ],
        out_specs=pl.BlockSpec((ROW_BLK, HIDDEN), lambda i: (i, 0)),
        out_shape=jax.ShapeDtypeStruct((ROWS_PAD, HIDDEN), jnp.float32),
    )(x, W1)


def _dinv_from(degp_ref):
    deg = degp_ref[0, :, 0:1] + degp_ref[1, :, 0:1] + 1.0
    return lax.rsqrt(deg)


def _tc_scale_body(degp_ref, mm1_ref, hp1_ref):
    hp1_ref[...] = mm1_ref[...] * _dinv_from(degp_ref)


def _tc_scale(degp, mm1):
    return pl.pallas_call(
        _tc_scale_body,
        grid=(N_PAD_BLK,),
        in_specs=[
            pl.BlockSpec((NC, PAD_BLK, 8), lambda i: (0, i, 0)),
            pl.BlockSpec((PAD_BLK, HIDDEN), lambda i: (i, 0)),
        ],
        out_specs=pl.BlockSpec((PAD_BLK, HIDDEN), lambda i: (i, 0)),
        out_shape=jax.ShapeDtypeStruct((ROWS_PAD, HIDDEN), jnp.float32),
    )(degp, mm1)


def _tc_b_body(degp_ref, hp1_ref, sp_ref, b_ref, w2_ref, hp2_ref):
    dinv = _dinv_from(degp_ref)
    h = (sp_ref[0] + sp_ref[1] + hp1_ref[...]) * dinv + b_ref[...]
    h = jnp.maximum(h, 0.0)
    hp2_ref[...] = jnp.dot(h, w2_ref[...],
                           preferred_element_type=jnp.float32) * dinv


def _tc_b(degp, hp1, sp, b1, W2):
    return pl.pallas_call(
        _tc_b_body,
        grid=(N_BLK,),
        in_specs=[
            pl.BlockSpec((NC, ROW_BLK, 8), lambda i: (0, i, 0)),
            pl.BlockSpec((ROW_BLK, HIDDEN), lambda i: (i, 0)),
            pl.BlockSpec((NC, ROW_BLK, HIDDEN), lambda i: (0, i, 0)),
            pl.BlockSpec((1, HIDDEN), lambda i: (0, 0)),
            pl.BlockSpec((HIDDEN, HIDDEN), lambda i: (0, 0)),
        ],
        out_specs=pl.BlockSpec((ROW_BLK, HIDDEN), lambda i: (i, 0)),
        out_shape=jax.ShapeDtypeStruct((ROWS_PAD, HIDDEN), jnp.float32),
    )(degp, hp1, sp, b1, W2)


def _tc_c_body(degp_ref, sp_ref, hp_ref, b_ref, batch_ref, wlin_ref,
               blin_ref, out_ref, pooled_scr, counts_scr):
    i = pl.program_id(0)

    @pl.when(i == 0)
    def _init():
        pooled_scr[...] = jnp.zeros_like(pooled_scr)
        counts_scr[...] = jnp.zeros_like(counts_scr)

    dinv = _dinv_from(degp_ref)
    h = (sp_ref[0] + sp_ref[1] + hp_ref[...]) * dinv + b_ref[...]
    h = jnp.maximum(h, 0.0)
    gids = lax.broadcasted_iota(jnp.int32, (ROW_BLK, N_GRAPHS), 1
                                ).astype(jnp.float32)
    onehot = jnp.where(batch_ref[...] == gids, 1.0, 0.0)
    pooled_scr[...] += lax.dot_general(
        onehot, h, (((0,), (0,)), ((), ())),
        preferred_element_type=jnp.float32)
    counts_scr[...] += lax.dot_general(
        onehot, jnp.ones((ROW_BLK, N_GRAPHS), jnp.float32),
        (((0,), (0,)), ((), ())), preferred_element_type=jnp.float32)

    @pl.when(i == N_BLK - 1)
    def _fin():
        counts = jnp.maximum(counts_scr[...][:, 0:HIDDEN], 1.0)
        pooled = pooled_scr[...] / counts
        out_ref[...] = jnp.dot(pooled, wlin_ref[...],
                               preferred_element_type=jnp.float32) + blin_ref[...]


def _tc_c(degp, sp, hp2, b2, batchf, Wlin, blin):
    return pl.pallas_call(
        _tc_c_body,
        grid=(N_BLK,),
        in_specs=[
            pl.BlockSpec((NC, ROW_BLK, 8), lambda i: (0, i, 0)),
            pl.BlockSpec((NC, ROW_BLK, HIDDEN), lambda i: (0, i, 0)),
            pl.BlockSpec((ROW_BLK, HIDDEN), lambda i: (i, 0)),
            pl.BlockSpec((1, HIDDEN), lambda i: (0, 0)),
            pl.BlockSpec((ROW_BLK, 1), lambda i: (i, 0)),
            pl.BlockSpec((HIDDEN, OUT_DIM), lambda i: (0, 0)),
            pl.BlockSpec((1, OUT_DIM), lambda i: (0, 0)),
        ],
        out_specs=pl.BlockSpec((N_GRAPHS, OUT_DIM), lambda i: (0, 0)),
        out_shape=jax.ShapeDtypeStruct((N_GRAPHS, OUT_DIM), jnp.float32),
        scratch_shapes=[
            pltpu.VMEM((N_GRAPHS, N_GRAPHS), jnp.float32),
            pltpu.VMEM((N_GRAPHS, N_GRAPHS), jnp.float32),
        ],
    )(degp, sp, hp2, b2, batchf, Wlin, blin)


@jax.jit
def kernel(x, edge_index, batch, W1, b1, W2, b2, Wlin, blin):
    srcp = edge_index[0].astype(jnp.int32).reshape(NW, CPT, CH)
    dstp = edge_index[1].astype(jnp.int32).reshape(NW, CPT, CH)

    zeros16 = jnp.zeros((RPT, 8), jnp.float32)
    zeros64 = jnp.zeros((RPT, HIDDEN), jnp.float32)
    ones_rows = jnp.ones((CH, 8), jnp.float32)

    mm1 = _tc_mm1(x, W1)
    degp = _deg_kernel(dstp, zeros16, ones_rows)
    hp1 = _tc_scale(degp, mm1)
    sp1 = _prop_kernel(hp1, srcp, dstp, zeros64)
    hp2 = _tc_b(degp, hp1, sp1, b1.reshape(1, HIDDEN), W2)
    sp2 = _prop_kernel(hp2, srcp, dstp, zeros64)
    batchf = batch.astype(jnp.float32).reshape(N_NODES, 1)
    logits = _tc_c(degp, sp2, hp2, b2.reshape(1, HIDDEN), batchf,
                   Wlin, blin.reshape(1, OUT_DIM))
    return logits
